# baseline (speedup 1.0000x reference)
import jax
import jax.numpy as jnp
from jax import lax
from jax.experimental import pallas as pl
from jax.experimental.pallas import tpu as pltpu
from jax.experimental.pallas import tpu_sc as plsc

N = 6_400_000
NUM_ELEMENTS = 89
TBL = 96
N_GRAPHS = 4096
NC = 2
NS = 16
NW = NC * NS
L = 16
PER_W = N // NW
CHUNK = 4000
N_CHUNKS = PER_W // CHUNK
LSTRIDE = CHUNK // L
ROWL = N_GRAPHS + 17
CROWS = 8
CSZ = (CROWS - 1) * ROWL + N_GRAPHS + 9
UNROLL = 125
GPW = N_GRAPHS // NW

_mesh = plsc.VectorSubcoreMesh(core_axis_name="c", subcore_axis_name="s")
_cparams = pltpu.CompilerParams(needs_layout_passes=False)


def _seg_body(en_hbm, sp_hbm, g_hbm, tbl_hbm,
              psums_hbm, pcnts_hbm,
              en0, sp0, g0, en1, sp1, g1,
              tbl_v, sums_acc, cnts_acc,
              sem0, sem1):
    wid = lax.axis_index("s") * NC + lax.axis_index("c")
    base = wid * PER_W

    pltpu.sync_copy(tbl_hbm, tbl_v)

    def start(ci, en_b, sp_b, g_b, sem):
        off = base + ci * CHUNK
        pltpu.make_async_copy(en_hbm.at[pl.ds(off, CHUNK)], en_b, sem).start()
        pltpu.make_async_copy(sp_hbm.at[pl.ds(off, CHUNK)], sp_b, sem).start()
        pltpu.make_async_copy(g_hbm.at[pl.ds(off, CHUNK)], g_b, sem).start()

    def wait(en_b, sp_b, g_b, sem):
        pltpu.make_async_copy(en_hbm.at[pl.ds(0, CHUNK)], en_b, sem).wait()
        pltpu.make_async_copy(sp_hbm.at[pl.ds(0, CHUNK)], sp_b, sem).wait()
        pltpu.make_async_copy(g_hbm.at[pl.ds(0, CHUNK)], g_b, sem).wait()

    start(0, en0, sp0, g0, sem0)
    start(1, en1, sp1, g1, sem1)

    zeros = jnp.zeros((16,), jnp.float32)

    def zz(i, carry):
        sums_acc[pl.ds(i * 16, 16)] = zeros
        cnts_acc[pl.ds(i * 16, 16)] = zeros
        return carry

    lax.fori_loop(0, CSZ // 16, zz, 0)

    ones = jnp.ones((16,), jnp.float32)
    lanes = lax.iota(jnp.int32, 16)
    sidx = lanes * LSTRIDE
    crow = (lanes & (CROWS - 1)) * ROWL

    def compute(en_b, sp_b, g_b):
        @plsc.parallel_loop(0, LSTRIDE, unroll=UNROLL)
        def _(v):
            idx = sidx + jnp.full((16,), v, jnp.int32)
            en = plsc.load_gather(en_b, [idx])
            sp = plsc.load_gather(sp_b, [idx])
            g = plsc.load_gather(g_b, [idx])
            w = plsc.load_gather(tbl_v, [sp * L + lanes])
            sc = plsc.bitcast(w & jnp.int32(-65536), jnp.float32)
            sh = plsc.bitcast(w << 16, jnp.float32)
            e = en * sc + sh
            a = crow + g
            plsc.addupdate_scatter(sums_acc, [a], e)
            plsc.addupdate_scatter(cnts_acc, [a], ones)

    def pair_body(p, carry):
        wait(en0, sp0, g0, sem0)
        compute(en0, sp0, g0)

        @pl.when(2 * p + 2 < N_CHUNKS)
        def _():
            start(2 * p + 2, en0, sp0, g0, sem0)

        wait(en1, sp1, g1, sem1)
        compute(en1, sp1, g1)

        @pl.when(2 * p + 3 < N_CHUNKS)
        def _():
            start(2 * p + 3, en1, sp1, g1, sem1)

        return carry

    lax.fori_loop(0, N_CHUNKS // 2, pair_body, 0)

    @plsc.parallel_loop(0, N_GRAPHS // 16, unroll=4)
    def _(k):
        b = k * 16
        s = sums_acc[pl.ds(b, 16)]
        for l in range(1, CROWS):
            s = s + sums_acc[pl.ds(l * ROWL + b, 16)]
        c = cnts_acc[pl.ds(b, 16)]
        for l in range(1, CROWS):
            c = c + cnts_acc[pl.ds(l * ROWL + b, 16)]
        sums_acc[pl.ds(b, 16)] = s
        cnts_acc[pl.ds(b, 16)] = c

    pltpu.sync_copy(sums_acc.at[pl.ds(0, N_GRAPHS)], psums_hbm.at[wid])
    pltpu.sync_copy(cnts_acc.at[pl.ds(0, N_GRAPHS)], pcnts_hbm.at[wid])


def _comb_body(psums_hbm, pcnts_hbm, out_hbm, sbuf, cbuf, obuf):
    wid = lax.axis_index("s") * NC + lax.axis_index("c")
    g0 = wid * GPW

    pltpu.sync_copy(psums_hbm.at[:, pl.ds(g0, GPW)], sbuf)
    pltpu.sync_copy(pcnts_hbm.at[:, pl.ds(g0, GPW)], cbuf)

    def vbody(v, carry):
        sl = pl.ds(v * 16, 16)

        def rbody(r, acc):
            return (acc[0] + sbuf[r, sl], acc[1] + cbuf[r, sl])

        ssum, csum = lax.fori_loop(
            0, NW, rbody,
            (jnp.zeros((16,), jnp.float32), jnp.zeros((16,), jnp.float32)))
        obuf[sl] = ssum / jnp.maximum(csum, 1.0)
        return carry

    lax.fori_loop(0, GPW // 16, vbody, 0)
    pltpu.sync_copy(obuf, out_hbm.at[pl.ds(g0, GPW)])


_seg = pl.kernel(
    _seg_body,
    mesh=_mesh,
    compiler_params=_cparams,
    out_type=(
        jax.ShapeDtypeStruct((NW, N_GRAPHS), jnp.float32),
        jax.ShapeDtypeStruct((NW, N_GRAPHS), jnp.float32),
    ),
    scratch_types=[
        pltpu.VMEM((CHUNK,), jnp.float32),
        pltpu.VMEM((CHUNK,), jnp.int32),
        pltpu.VMEM((CHUNK,), jnp.int32),
        pltpu.VMEM((CHUNK,), jnp.float32),
        pltpu.VMEM((CHUNK,), jnp.int32),
        pltpu.VMEM((CHUNK,), jnp.int32),
        pltpu.VMEM((TBL * L,), jnp.int32),
        pltpu.VMEM((CSZ,), jnp.float32),
        pltpu.VMEM((CSZ,), jnp.float32),
        pltpu.SemaphoreType.DMA,
        pltpu.SemaphoreType.DMA,
    ],
)

_comb = pl.kernel(
    _comb_body,
    mesh=_mesh,
    compiler_params=_cparams,
    out_type=jax.ShapeDtypeStruct((N_GRAPHS,), jnp.float32),
    scratch_types=[
        pltpu.VMEM((NW, GPW), jnp.float32),
        pltpu.VMEM((NW, GPW), jnp.float32),
        pltpu.VMEM((GPW,), jnp.float32),
    ],
)


def kernel(energies, species, graph_i, n_graphs, scale, shift):
    del n_graphs
    pad = jnp.zeros((TBL - NUM_ELEMENTS,), jnp.float32)
    scale_p = jnp.concatenate([scale.astype(jnp.float32), pad])
    shift_p = jnp.concatenate([shift.astype(jnp.float32), pad])
    hi = scale_p.astype(jnp.bfloat16).view(jnp.uint16).astype(jnp.uint32)
    lo = shift_p.astype(jnp.bfloat16).view(jnp.uint16).astype(jnp.uint32)
    tbl = jnp.repeat(((hi << 16) | lo).view(jnp.int32), L)
    psums, pcnts = _seg(
        energies.astype(jnp.float32),
        species.astype(jnp.int32),
        graph_i.astype(jnp.int32),
        tbl)
    out = _comb(psums, pcnts)
    return out[:, None]


# speedup vs baseline: 1.3026x; 1.3026x over previous
import jax
import jax.numpy as jnp
from jax import lax
from jax.experimental import pallas as pl
from jax.experimental.pallas import tpu as pltpu
from jax.experimental.pallas import tpu_sc as plsc

N = 6_400_000
NUM_ELEMENTS = 89
TBL = 96
N_GRAPHS = 4096
NC = 2
NS = 16
NW = NC * NS
L = 16
PER_W = N // NW
CHUNK = 4000
N_CHUNKS = PER_W // CHUNK
LSTRIDE = CHUNK // L
ROWL = N_GRAPHS + 17
CROWS = 8
CSZ = (CROWS - 1) * ROWL + N_GRAPHS + 9
UNROLL = 50
GPW = N_GRAPHS // NW

_mesh = plsc.VectorSubcoreMesh(core_axis_name="c", subcore_axis_name="s")
_cparams = pltpu.CompilerParams(needs_layout_passes=False)


def _seg_body(en_hbm, sp_hbm, g_hbm, tbl_hbm,
              psums_hbm, pcnts_hbm,
              en0, sp0, g0, en1, sp1, g1,
              tbl_v, sums_acc, cnts_acc,
              sem0, sem1):
    wid = lax.axis_index("s") * NC + lax.axis_index("c")
    base = wid * PER_W

    pltpu.sync_copy(tbl_hbm, tbl_v)

    def start(ci, en_b, sp_b, g_b, sem):
        off = base + ci * CHUNK
        pltpu.make_async_copy(en_hbm.at[pl.ds(off, CHUNK)], en_b, sem).start()
        pltpu.make_async_copy(sp_hbm.at[pl.ds(off, CHUNK)], sp_b, sem).start()
        pltpu.make_async_copy(g_hbm.at[pl.ds(off, CHUNK)], g_b, sem).start()

    def wait(en_b, sp_b, g_b, sem):
        pltpu.make_async_copy(en_hbm.at[pl.ds(0, CHUNK)], en_b, sem).wait()
        pltpu.make_async_copy(sp_hbm.at[pl.ds(0, CHUNK)], sp_b, sem).wait()
        pltpu.make_async_copy(g_hbm.at[pl.ds(0, CHUNK)], g_b, sem).wait()

    start(0, en0, sp0, g0, sem0)
    start(1, en1, sp1, g1, sem1)

    zeros = jnp.zeros((16,), jnp.float32)

    def zz(i, carry):
        sums_acc[pl.ds(i * 16, 16)] = zeros
        cnts_acc[pl.ds(i * 16, 16)] = zeros
        return carry

    lax.fori_loop(0, CSZ // 16, zz, 0)

    ones = jnp.ones((16,), jnp.float32)
    lanes = lax.iota(jnp.int32, 16)
    sidx = lanes * LSTRIDE
    crow = (lanes & (CROWS - 1)) * ROWL

    def compute(en_b, sp_b, g_b):
        @plsc.parallel_loop(0, LSTRIDE, unroll=UNROLL)
        def _(v):
            idx = sidx + jnp.full((16,), v, jnp.int32)
            en = plsc.load_gather(en_b, [idx])
            sp = plsc.load_gather(sp_b, [idx])
            g = plsc.load_gather(g_b, [idx])
            w = plsc.load_gather(tbl_v, [sp * L + lanes])
            sc = plsc.bitcast(w & jnp.int32(-65536), jnp.float32)
            sh = plsc.bitcast(w << 16, jnp.float32)
            e = en * sc + sh
            a = crow + g
            plsc.addupdate_scatter(sums_acc, [a], e)
            plsc.addupdate_scatter(cnts_acc, [a], ones)

    def pair_body(p, carry):
        wait(en0, sp0, g0, sem0)
        compute(en0, sp0, g0)

        @pl.when(2 * p + 2 < N_CHUNKS)
        def _():
            start(2 * p + 2, en0, sp0, g0, sem0)

        wait(en1, sp1, g1, sem1)
        compute(en1, sp1, g1)

        @pl.when(2 * p + 3 < N_CHUNKS)
        def _():
            start(2 * p + 3, en1, sp1, g1, sem1)

        return carry

    lax.fori_loop(0, N_CHUNKS // 2, pair_body, 0)

    @plsc.parallel_loop(0, N_GRAPHS // 16, unroll=4)
    def _(k):
        b = k * 16
        s = sums_acc[pl.ds(b, 16)]
        for l in range(1, CROWS):
            s = s + sums_acc[pl.ds(l * ROWL + b, 16)]
        c = cnts_acc[pl.ds(b, 16)]
        for l in range(1, CROWS):
            c = c + cnts_acc[pl.ds(l * ROWL + b, 16)]
        sums_acc[pl.ds(b, 16)] = s
        cnts_acc[pl.ds(b, 16)] = c

    pltpu.sync_copy(sums_acc.at[pl.ds(0, N_GRAPHS)], psums_hbm.at[wid])
    pltpu.sync_copy(cnts_acc.at[pl.ds(0, N_GRAPHS)], pcnts_hbm.at[wid])


def _comb_body(psums_hbm, pcnts_hbm, out_hbm, sbuf, cbuf, obuf):
    wid = lax.axis_index("s") * NC + lax.axis_index("c")
    g0 = wid * GPW

    pltpu.sync_copy(psums_hbm.at[:, pl.ds(g0, GPW)], sbuf)
    pltpu.sync_copy(pcnts_hbm.at[:, pl.ds(g0, GPW)], cbuf)

    def vbody(v, carry):
        sl = pl.ds(v * 16, 16)

        def rbody(r, acc):
            return (acc[0] + sbuf[r, sl], acc[1] + cbuf[r, sl])

        ssum, csum = lax.fori_loop(
            0, NW, rbody,
            (jnp.zeros((16,), jnp.float32), jnp.zeros((16,), jnp.float32)))
        obuf[sl] = ssum / jnp.maximum(csum, 1.0)
        return carry

    lax.fori_loop(0, GPW // 16, vbody, 0)
    pltpu.sync_copy(obuf, out_hbm.at[pl.ds(g0, GPW)])


_seg = pl.kernel(
    _seg_body,
    mesh=_mesh,
    compiler_params=_cparams,
    out_type=(
        jax.ShapeDtypeStruct((NW, N_GRAPHS), jnp.float32),
        jax.ShapeDtypeStruct((NW, N_GRAPHS), jnp.float32),
    ),
    scratch_types=[
        pltpu.VMEM((CHUNK,), jnp.float32),
        pltpu.VMEM((CHUNK,), jnp.int32),
        pltpu.VMEM((CHUNK,), jnp.int32),
        pltpu.VMEM((CHUNK,), jnp.float32),
        pltpu.VMEM((CHUNK,), jnp.int32),
        pltpu.VMEM((CHUNK,), jnp.int32),
        pltpu.VMEM((TBL * L,), jnp.int32),
        pltpu.VMEM((CSZ,), jnp.float32),
        pltpu.VMEM((CSZ,), jnp.float32),
        pltpu.SemaphoreType.DMA,
        pltpu.SemaphoreType.DMA,
    ],
)

_comb = pl.kernel(
    _comb_body,
    mesh=_mesh,
    compiler_params=_cparams,
    out_type=jax.ShapeDtypeStruct((N_GRAPHS,), jnp.float32),
    scratch_types=[
        pltpu.VMEM((NW, GPW), jnp.float32),
        pltpu.VMEM((NW, GPW), jnp.float32),
        pltpu.VMEM((GPW,), jnp.float32),
    ],
)


def kernel(energies, species, graph_i, n_graphs, scale, shift):
    del n_graphs
    pad = jnp.zeros((TBL - NUM_ELEMENTS,), jnp.float32)
    scale_p = jnp.concatenate([scale.astype(jnp.float32), pad])
    shift_p = jnp.concatenate([shift.astype(jnp.float32), pad])
    hi = scale_p.astype(jnp.bfloat16).view(jnp.uint16).astype(jnp.uint32)
    lo = shift_p.astype(jnp.bfloat16).view(jnp.uint16).astype(jnp.uint32)
    tbl = jnp.repeat(((hi << 16) | lo).view(jnp.int32), L)
    psums, pcnts = _seg(
        energies.astype(jnp.float32),
        species.astype(jnp.int32),
        graph_i.astype(jnp.int32),
        tbl)
    out = _comb(psums, pcnts)
    return out[:, None]
